# Initial kernel scaffold; baseline (speedup 1.0000x reference)
import jax
import jax.numpy as jnp
from jax.experimental import pallas as pl


def kernel(edge_index, node_attr, W, b):
    raise NotImplementedError("write your pallas kernel here")


# speedup vs baseline: 4.0651x; 4.0651x over previous
import functools

import jax
import jax.numpy as jnp
from jax import lax
from jax.experimental import pallas as pl
from jax.experimental.pallas import tpu as pltpu
from jax.experimental.pallas import tpu_sc as plsc

NC = 2
NS = 16
NW = NC * NS
CH = 128
IB = 8
EC = 2048


def _round_up(x, m):
    return (x + m - 1) // m * m


def _make_sc_aggregate(n_pad, k_chunks, d):
    stripe = n_pad // NS
    ZB = stripe // CH
    mesh = plsc.VectorSubcoreMesh(core_axis_name="c", subcore_axis_name="s")

    @functools.partial(
        pl.kernel,
        out_type=jax.ShapeDtypeStruct((NC, n_pad, d), jnp.float32),
        mesh=mesh,
        scratch_types=[
            pltpu.VMEM((IB, CH), jnp.int32),
            pltpu.VMEM((IB, CH), jnp.int32),
            pltpu.VMEM((ZB, CH), jnp.int32),
            pltpu.VMEM((CH, d), jnp.float32),
            pltpu.VMEM_SHARED((n_pad, d), jnp.float32),
            pltpu.SemaphoreType.DMA,
        ],
    )
    def sc_aggregate(src_hbm, dst_hbm, node_hbm, iota_hbm, zrows_hbm,
                     sums_out, src_v, dst_v, iota_v, rows_v, acc_sh, sem):
        c = lax.axis_index("c")
        s = lax.axis_index("s")
        wid = s * NC + c

        pltpu.sync_copy(iota_hbm.at[s], iota_v)
        pltpu.sync_copy(zrows_hbm, rows_v)
        for z in range(ZB):
            pltpu.sync_copy(rows_v, acc_sh.at[iota_v.at[z]])
        plsc.subcore_barrier()

        def group(g, carry):
            pltpu.sync_copy(src_hbm.at[wid, pl.ds(g * IB, IB)], src_v)
            pltpu.sync_copy(dst_hbm.at[wid, pl.ds(g * IB, IB)], dst_v)
            for j in range(IB):
                pltpu.async_copy(node_hbm.at[src_v.at[j]], rows_v, sem).wait()
                pltpu.sync_copy(rows_v, acc_sh.at[dst_v.at[j]], add=True)
            return carry

        lax.fori_loop(0, k_chunks // IB, group, 0)
        plsc.subcore_barrier()

        for z in range(ZB):
            r0 = s * stripe + z * CH
            pltpu.async_copy(acc_sh.at[iota_v.at[z]], rows_v, sem).wait()
            pltpu.sync_copy(rows_v, sums_out.at[c, pl.ds(r0, CH)])

    return sc_aggregate


def _tc_counts(dst2d, n_pad, d):
    nq = n_pad // d
    nblk = dst2d.shape[0]
    dst2d = dst2d.reshape(nblk, 1, EC)

    def body(d_ref, o_ref):
        @pl.when(pl.program_id(0) == 0)
        def _():
            o_ref[...] = jnp.zeros_like(o_ref)

        dv = d_ref[0, 0]
        q = (dv >> (d.bit_length() - 1))[:, None]
        r = (dv & (d - 1))[:, None]
        m = (q == lax.broadcasted_iota(jnp.int32, (1, nq), 1)
             ).astype(jnp.float32)
        p = (r == lax.broadcasted_iota(jnp.int32, (1, d), 1)
             ).astype(jnp.float32)
        o_ref[...] += lax.dot_general(
            m, p, (((0,), (0,)), ((), ())),
            preferred_element_type=jnp.float32)

    return pl.pallas_call(
        body,
        out_shape=jax.ShapeDtypeStruct((nq, d), jnp.float32),
        grid=(nblk,),
        in_specs=[pl.BlockSpec((1, 1, EC), lambda i: (i, 0, 0))],
        out_specs=pl.BlockSpec((nq, d), lambda i: (0, 0)),
    )(dst2d)


def _tc_combine(x_pad, sums, cnts2d, W, b, n_pad, d, rows_blk):
    def body(x_ref, s_ref, c_ref, w_ref, b_ref, o_ref):
        cnt = jnp.maximum(c_ref[0, 0], 1.0)
        agg = (s_ref[0] + s_ref[1]) / cnt[:, None]
        y = (jnp.dot(x_ref[...], w_ref[:d, :],
                     preferred_element_type=jnp.float32,
                     precision=lax.Precision.HIGHEST)
             + jnp.dot(agg, w_ref[d:, :],
                       preferred_element_type=jnp.float32,
                       precision=lax.Precision.HIGHEST)
             + b_ref[...])
        o_ref[...] = jnp.maximum(y, 0.0)

    return pl.pallas_call(
        body,
        out_shape=jax.ShapeDtypeStruct((n_pad, d), jnp.float32),
        grid=(n_pad // rows_blk,),
        in_specs=[
            pl.BlockSpec((rows_blk, d), lambda i: (i, 0)),
            pl.BlockSpec((NC, rows_blk, d), lambda i: (0, i, 0)),
            pl.BlockSpec((1, 1, rows_blk), lambda i: (i, 0, 0)),
            pl.BlockSpec((2 * d, d), lambda i: (0, 0)),
            pl.BlockSpec((1, d), lambda i: (0, 0)),
        ],
        out_specs=pl.BlockSpec((rows_blk, d), lambda i: (i, 0)),
    )(x_pad, sums, cnts2d, W, b.reshape(1, d))


def kernel(edge_index, node_attr, W, b):
    n, d = node_attr.shape
    e = edge_index.shape[1]

    e_pad = _round_up(e, NW * CH * IB)
    k_chunks = e_pad // (NW * CH)
    n_pad = _round_up(n + 1, 2048)
    rows_blk = 512

    src = edge_index[0].astype(jnp.int32)
    dst = edge_index[1].astype(jnp.int32)
    src3 = jnp.pad(src, (0, e_pad - e)).reshape(NW, k_chunks, CH)
    dst3 = jnp.pad(dst, (0, e_pad - e),
                   constant_values=n).reshape(NW, k_chunks, CH)

    iota = jnp.arange(n_pad, dtype=jnp.int32).reshape(NS, n_pad // NS // CH, CH)
    zrows = jnp.zeros((CH, d), jnp.float32)

    sc_aggregate = _make_sc_aggregate(n_pad, k_chunks, d)
    sums = sc_aggregate(src3, dst3, node_attr, iota, zrows)

    e_pad2 = _round_up(e, EC)
    dst2d = jnp.pad(dst, (0, e_pad2 - e),
                    constant_values=n).reshape(e_pad2 // EC, EC)
    cnts = _tc_counts(dst2d, n_pad, d)
    cnts2d = cnts.reshape(n_pad // rows_blk, 1, rows_blk)

    x_pad = jnp.pad(node_attr, ((0, n_pad - n), (0, 0)))
    out = _tc_combine(x_pad, sums, cnts2d, W, b, n_pad, d, rows_blk)
    return out[:n]


# speedup vs baseline: 4.4953x; 1.1058x over previous
import functools

import jax
import jax.numpy as jnp
from jax import lax
from jax.experimental import pallas as pl
from jax.experimental.pallas import tpu as pltpu
from jax.experimental.pallas import tpu_sc as plsc

NC = 2
NS = 16
NW = NC * NS
CH = 128
IB = 16
EC = 2048


def _round_up(x, m):
    return (x + m - 1) // m * m


def _make_sc_aggregate(n_pad, k_chunks, d):
    stripe = n_pad // NS
    ZB = stripe // CH
    mesh = plsc.VectorSubcoreMesh(core_axis_name="c", subcore_axis_name="s")

    @functools.partial(
        pl.kernel,
        out_type=jax.ShapeDtypeStruct((NC, n_pad, d), jnp.float32),
        mesh=mesh,
        scratch_types=[
            pltpu.VMEM((IB, CH), jnp.int32),
            pltpu.VMEM((IB, CH), jnp.int32),
            pltpu.VMEM((ZB, CH), jnp.int32),
            pltpu.VMEM((CH, d), jnp.float32),
            pltpu.VMEM((CH, d), jnp.float32),
            pltpu.VMEM_SHARED((n_pad, d), jnp.float32),
            pltpu.SemaphoreType.DMA,
            pltpu.SemaphoreType.DMA,
        ],
    )
    def sc_aggregate(src_hbm, dst_hbm, node_hbm, iota_hbm, zrows_hbm,
                     sums_out, src_v, dst_v, iota_v, rows_a, rows_b,
                     acc_sh, sem_a, sem_b):
        c = lax.axis_index("c")
        s = lax.axis_index("s")
        wid = s * NC + c

        pltpu.sync_copy(iota_hbm.at[s], iota_v)
        pltpu.sync_copy(zrows_hbm, rows_a)
        for z in range(ZB):
            pltpu.sync_copy(rows_a, acc_sh.at[iota_v.at[z]])
        plsc.subcore_barrier()

        def group(g, carry):
            pltpu.sync_copy(src_hbm.at[wid, pl.ds(g * IB, IB)], src_v)
            pltpu.sync_copy(dst_hbm.at[wid, pl.ds(g * IB, IB)], dst_v)
            pltpu.async_copy(node_hbm.at[src_v.at[0]], rows_a, sem_a)
            for j in range(IB):
                cur, sc = (rows_a, sem_a) if j % 2 == 0 else (rows_b, sem_b)
                nxt, sn = (rows_b, sem_b) if j % 2 == 0 else (rows_a, sem_a)
                pltpu.make_async_copy(node_hbm.at[src_v.at[j]], cur, sc).wait()
                if j + 1 < IB:
                    pltpu.async_copy(node_hbm.at[src_v.at[j + 1]], nxt, sn)
                pltpu.sync_copy(cur, acc_sh.at[dst_v.at[j]], add=True)
            return carry

        lax.fori_loop(0, k_chunks // IB, group, 0)
        plsc.subcore_barrier()

        for z in range(ZB):
            r0 = s * stripe + z * CH
            pltpu.async_copy(acc_sh.at[iota_v.at[z]], rows_a, sem_a).wait()
            pltpu.sync_copy(rows_a, sums_out.at[c, pl.ds(r0, CH)])

    return sc_aggregate


def _tc_counts(dst2d, n_pad, d):
    nq = n_pad // d
    nblk = dst2d.shape[0]
    dst2d = dst2d.reshape(nblk, 1, EC)

    def body(d_ref, o_ref):
        @pl.when(pl.program_id(0) == 0)
        def _():
            o_ref[...] = jnp.zeros_like(o_ref)

        dv = d_ref[0, 0]
        q = (dv >> (d.bit_length() - 1))[:, None]
        r = (dv & (d - 1))[:, None]
        m = (q == lax.broadcasted_iota(jnp.int32, (1, nq), 1)
             ).astype(jnp.float32)
        p = (r == lax.broadcasted_iota(jnp.int32, (1, d), 1)
             ).astype(jnp.float32)
        o_ref[...] += lax.dot_general(
            m, p, (((0,), (0,)), ((), ())),
            preferred_element_type=jnp.float32)

    return pl.pallas_call(
        body,
        out_shape=jax.ShapeDtypeStruct((nq, d), jnp.float32),
        grid=(nblk,),
        in_specs=[pl.BlockSpec((1, 1, EC), lambda i: (i, 0, 0))],
        out_specs=pl.BlockSpec((nq, d), lambda i: (0, 0)),
    )(dst2d)


def _tc_combine(x_pad, sums, cnts2d, W, b, n_pad, d, rows_blk):
    def body(x_ref, s_ref, c_ref, w_ref, b_ref, o_ref):
        cnt = jnp.maximum(c_ref[0, 0], 1.0)
        agg = (s_ref[0] + s_ref[1]) / cnt[:, None]
        y = (jnp.dot(x_ref[...], w_ref[:d, :],
                     preferred_element_type=jnp.float32,
                     precision=lax.Precision.HIGHEST)
             + jnp.dot(agg, w_ref[d:, :],
                       preferred_element_type=jnp.float32,
                       precision=lax.Precision.HIGHEST)
             + b_ref[...])
        o_ref[...] = jnp.maximum(y, 0.0)

    return pl.pallas_call(
        body,
        out_shape=jax.ShapeDtypeStruct((n_pad, d), jnp.float32),
        grid=(n_pad // rows_blk,),
        in_specs=[
            pl.BlockSpec((rows_blk, d), lambda i: (i, 0)),
            pl.BlockSpec((NC, rows_blk, d), lambda i: (0, i, 0)),
            pl.BlockSpec((1, 1, rows_blk), lambda i: (i, 0, 0)),
            pl.BlockSpec((2 * d, d), lambda i: (0, 0)),
            pl.BlockSpec((1, d), lambda i: (0, 0)),
        ],
        out_specs=pl.BlockSpec((rows_blk, d), lambda i: (i, 0)),
    )(x_pad, sums, cnts2d, W, b.reshape(1, d))


def kernel(edge_index, node_attr, W, b):
    n, d = node_attr.shape
    e = edge_index.shape[1]

    e_pad = _round_up(e, NW * CH * IB)
    k_chunks = e_pad // (NW * CH)
    n_pad = _round_up(n + 1, 2048)
    rows_blk = 512

    src = edge_index[0].astype(jnp.int32)
    dst = edge_index[1].astype(jnp.int32)
    src3 = jnp.pad(src, (0, e_pad - e)).reshape(NW, k_chunks, CH)
    dst3 = jnp.pad(dst, (0, e_pad - e),
                   constant_values=n).reshape(NW, k_chunks, CH)

    iota = jnp.arange(n_pad, dtype=jnp.int32).reshape(NS, n_pad // NS // CH, CH)
    zrows = jnp.zeros((CH, d), jnp.float32)

    sc_aggregate = _make_sc_aggregate(n_pad, k_chunks, d)
    sums = sc_aggregate(src3, dst3, node_attr, iota, zrows)

    e_pad2 = _round_up(e, EC)
    dst2d = jnp.pad(dst, (0, e_pad2 - e),
                    constant_values=n).reshape(e_pad2 // EC, EC)
    cnts = _tc_counts(dst2d, n_pad, d)
    cnts2d = cnts.reshape(n_pad // rows_blk, 1, rows_blk)

    x_pad = jnp.pad(node_attr, ((0, n_pad - n), (0, 0)))
    out = _tc_combine(x_pad, sums, cnts2d, W, b, n_pad, d, rows_blk)
    return out[:n]
